# baseline (speedup 1.0000x reference)
import functools

import jax
import jax.numpy as jnp
from jax import lax
from jax.experimental import pallas as pl
from jax.experimental.pallas import tpu as pltpu
from jax.experimental.pallas import tpu_sc as plsc

NUM_WORKERS = 32
K = 8
TBLK = 256


@functools.partial(jax.jit, static_argnames=("vocab", "dim", "tail"))
def _sc_transpose(table_v4, table_tail, *, vocab, dim, tail):
    lanes = 2 * dim
    vmain = vocab - tail
    n_tiles = vmain // 128
    n_full = vmain // TBLK
    per_w = n_full // NUM_WORKERS
    rem = n_full - per_w * NUM_WORKERS
    out_rows = vocab * dim // lanes
    sub = dim // 8

    mesh = plsc.VectorSubcoreMesh(core_axis_name="c", subcore_axis_name="s")

    @functools.partial(
        pl.kernel,
        out_type=jax.ShapeDtypeStruct((out_rows * lanes,), jnp.float32),
        mesh=mesh,
        scratch_types=[
            pltpu.VMEM((sub, 2, 8, 128), jnp.float32),
            pltpu.VMEM((sub, 2, 8, 128), jnp.float32),
            pltpu.VMEM((TBLK // 2 * lanes,), jnp.float32),
            pltpu.VMEM((TBLK // 2 * lanes,), jnp.float32),
            pltpu.VMEM((dim, tail), jnp.float32),
            pltpu.SemaphoreType.DMA((2,)),
            pltpu.SemaphoreType.DMA((2,)),
        ],
        compiler_params=pltpu.CompilerParams(
            use_tc_tiling_on_sc=False, needs_layout_passes=False
        ),
    )
    def k(tv4_hbm, ttail_hbm, out_hbm, blk0, blk1, tp0, tp1, tail_v,
          lsem, wsem):
        wid = lax.axis_index("s") * 2 + lax.axis_index("c")
        start = wid * per_w + jnp.minimum(wid, rem)
        count = per_w + jnp.where(wid < rem, 1, 0)
        blks = (blk0, blk1)
        tps = (tp0, tp1)

        def load(i, p):
            c0 = (start + i) * 2
            pltpu.async_copy(
                tv4_hbm.at[:, pl.ds(c0, 2), :, :], blks[p], lsem.at[p]
            )

        def wait_load(p):
            pltpu.make_async_copy(
                tv4_hbm.at[:, pl.ds(0, 2), :, :], blks[p], lsem.at[p]
            ).wait()

        def wait_write(p):
            pltpu.make_async_copy(
                tps[p], out_hbm.at[pl.ds(0, TBLK // 2 * lanes)], wsem.at[p]
            ).wait()

        def transpose_block(p):
            blk, tp = blks[p], tps[p]
            for l0 in range(0, 128, 16):
                lvec = l0 + lax.iota(jnp.int32, 16)
                half = lax.shift_right_logical(lvec, 1)
                colp = (lvec & 1) * dim
                for j in range(2):
                    base = half * lanes + (j * (128 // 2) * lanes) + colp

                    def rbody(R, carry):
                        for r in range(8):
                            v = blk[R, j, r, pl.ds(l0, 16)]
                            plsc.store_scatter(tp, [base + (R * 8 + r)], v)
                        return carry

                    lax.fori_loop(0, sub, rbody, 0)

        def step(i, p):
            wait_load(p)

            @pl.when(i >= 2)
            def _():
                wait_write(p)

            transpose_block(p)
            j0 = (start + i) * (TBLK // 2 * lanes)
            pltpu.async_copy(
                tps[p], out_hbm.at[pl.ds(j0, TBLK // 2 * lanes)], wsem.at[p]
            )

            @pl.when(i + 2 < count)
            def _():
                load(i + 2, p)

        def body(i2, carry):
            step(2 * i2, 0)
            step(2 * i2 + 1, 1)
            return carry

        n2 = count // 2
        load(0, 0)

        @pl.when(count > 1)
        def _():
            load(1, 1)

        lax.fori_loop(0, n2, body, 0)

        @pl.when(count != 2 * n2)
        def _():
            step(count - 1, 0)

        @pl.when(count >= 1)
        def _():
            wait_write(0)

        @pl.when(count >= 2)
        def _():
            wait_write(1)

        if tail:
            @pl.when(wid == NUM_WORKERS - 1)
            def _():
                tok0 = n_full * TBLK
                pltpu.sync_copy(ttail_hbm, tail_v)
                for l0 in range(0, tail, 16):
                    lvec = l0 + lax.iota(jnp.int32, 16)
                    row_idx = lax.shift_right_logical(lvec, 1)
                    col_base = (lvec & 1) * dim

                    base = row_idx * lanes + col_base

                    def cbody(c, carry):
                        v = tail_v[c, pl.ds(l0, 16)]
                        cvec = jnp.full((16,), 0, jnp.int32) + c
                        plsc.store_scatter(tp0, [base + cvec], v)
                        return carry

                    lax.fori_loop(0, dim, cbody, 0)
                pltpu.sync_copy(
                    tp0.at[pl.ds(0, tail // 2 * lanes)],
                    out_hbm.at[pl.ds(tok0 // 2 * lanes, tail // 2 * lanes)],
                )

    return k(table_v4, table_tail)


@functools.partial(jax.jit, static_argnames=("batch", "seq", "dim"))
def _sc_gather(ids, table, *, batch, seq, dim):
    rows_per_w = batch // NUM_WORKERS
    n_groups = rows_per_w // K

    mesh = plsc.VectorSubcoreMesh(core_axis_name="c", subcore_axis_name="s")

    @functools.partial(
        pl.kernel,
        out_type=jax.ShapeDtypeStruct((batch * seq, 2, dim), jnp.float32),
        mesh=mesh,
        scratch_types=[
            pltpu.VMEM((K, seq), jnp.int32),
            pltpu.VMEM((K, seq, dim), jnp.float32),
            pltpu.SemaphoreType.DMA((K,)),
            pltpu.SemaphoreType.DMA((K,)),
            pltpu.SemaphoreType.DMA((K,)),
        ],
        compiler_params=pltpu.CompilerParams(use_tc_tiling_on_sc=False),
    )
    def k(ids_hbm, table_hbm, out_hbm, idx_v, rows_v, isem, gsem, ssem):
        wid = lax.axis_index("s") * 2 + lax.axis_index("c")
        b0 = wid * rows_per_w

        def body(g, carry):
            r0 = b0 + g * K
            for b in range(K):
                pltpu.async_copy(ids_hbm.at[r0 + b], idx_v.at[b], isem.at[b])
            for b in range(K):
                pltpu.make_async_copy(
                    ids_hbm.at[r0 + b], idx_v.at[b], isem.at[b]
                ).wait()
                pltpu.async_copy(
                    table_hbm.at[idx_v.at[b]], rows_v.at[b], gsem.at[b]
                )
            for b in range(K):
                pltpu.make_async_copy(
                    table_hbm.at[idx_v.at[b]], rows_v.at[b], gsem.at[b]
                ).wait()
                pltpu.async_copy(
                    rows_v.at[b],
                    out_hbm.at[pl.ds((r0 + b) * seq, seq), 0, :],
                    ssem.at[b],
                )
            for b in range(K):
                pltpu.make_async_copy(
                    rows_v.at[b],
                    out_hbm.at[pl.ds((r0 + b) * seq, seq), 0, :],
                    ssem.at[b],
                ).wait()
            return carry

        lax.fori_loop(0, n_groups, body, 0)

    return k(ids, table)


def kernel(input_ids, token_embed):
    batch, seq = input_ids.shape
    vocab, dim = token_embed.shape
    tail = vocab % 128
    vmain = vocab - tail
    table_t_full = lax.optimization_barrier(token_embed.T)
    table_v4 = (
        table_t_full[:, :vmain]
        .reshape(dim // 8, 8, vmain // 128, 128)
        .transpose(0, 2, 1, 3)
    )
    table_tail = table_t_full[:, vmain:]
    t1d = _sc_transpose(
        table_v4, table_tail, vocab=vocab, dim=dim, tail=tail
    )
    t_lin = t1d.reshape(vocab, dim)
    out2 = _sc_gather(input_ids, t_lin, batch=batch, seq=seq, dim=dim)
    return out2.reshape(batch, seq, 2 * dim)[..., :dim]


# speedup vs baseline: 1.8345x; 1.8345x over previous
import functools

import jax
import jax.numpy as jnp
from jax import lax
from jax.experimental import pallas as pl
from jax.experimental.pallas import tpu as pltpu
from jax.experimental.pallas import tpu_sc as plsc

NUM_WORKERS = 32
K = 8
TBLK = 256


@functools.partial(jax.jit, static_argnames=("vocab", "dim", "tail"))
def _sc_transpose(table_v4, table_tail, *, vocab, dim, tail):
    lanes = 2 * dim
    vmain = vocab - tail
    n_tiles = vmain // 128
    n_full = vmain // TBLK
    per_w = n_full // NUM_WORKERS
    rem = n_full - per_w * NUM_WORKERS
    out_rows = vocab * dim // lanes
    sub = dim // 8

    mesh = plsc.VectorSubcoreMesh(core_axis_name="c", subcore_axis_name="s")

    @functools.partial(
        pl.kernel,
        out_type=jax.ShapeDtypeStruct((out_rows * lanes,), jnp.float32),
        mesh=mesh,
        scratch_types=[
            pltpu.VMEM((sub, 2, 8, 128), jnp.float32),
            pltpu.VMEM((sub, 2, 8, 128), jnp.float32),
            pltpu.VMEM((TBLK // 2 * lanes,), jnp.float32),
            pltpu.VMEM((TBLK // 2 * lanes,), jnp.float32),
            pltpu.VMEM((dim, tail), jnp.float32),
            pltpu.SemaphoreType.DMA((2,)),
            pltpu.SemaphoreType.DMA((2,)),
        ],
        compiler_params=pltpu.CompilerParams(
            use_tc_tiling_on_sc=False, needs_layout_passes=False
        ),
    )
    def k(tv4_hbm, ttail_hbm, out_hbm, blk0, blk1, tp0, tp1, tail_v,
          lsem, wsem):
        wid = lax.axis_index("s") * 2 + lax.axis_index("c")
        start = wid * per_w + jnp.minimum(wid, rem)
        count = per_w + jnp.where(wid < rem, 1, 0)
        blks = (blk0, blk1)
        tps = (tp0, tp1)

        def load(i, p):
            c0 = (start + i) * 2
            pltpu.async_copy(
                tv4_hbm.at[:, pl.ds(c0, 2), :, :], blks[p], lsem.at[p]
            )

        def wait_load(p):
            pltpu.make_async_copy(
                tv4_hbm.at[:, pl.ds(0, 2), :, :], blks[p], lsem.at[p]
            ).wait()

        def wait_write(p):
            pltpu.make_async_copy(
                tps[p], out_hbm.at[pl.ds(0, TBLK // 2 * lanes)], wsem.at[p]
            ).wait()

        def transpose_block(p):
            blk, tp = blks[p], tps[p]
            for l0 in range(0, 128, 16):
                lvec = l0 + lax.iota(jnp.int32, 16)
                half = lax.shift_right_logical(lvec, 1)
                colp = (lvec & 1) * dim
                for j in range(2):
                    base = half * lanes + (j * (128 // 2) * lanes) + colp

                    def rbody(R, carry):
                        for r in range(8):
                            v = blk[R, j, r, pl.ds(l0, 16)]
                            plsc.store_scatter(tp, [base + (R * 8 + r)], v)
                        return carry

                    lax.fori_loop(0, sub, rbody, 0)

        def step(i, p):
            wait_load(p)

            @pl.when(i >= 2)
            def _():
                wait_write(p)

            transpose_block(p)
            j0 = (start + i) * (TBLK // 2 * lanes)
            pltpu.async_copy(
                tps[p], out_hbm.at[pl.ds(j0, TBLK // 2 * lanes)], wsem.at[p]
            )

            @pl.when(i + 2 < count)
            def _():
                load(i + 2, p)

        def body(i2, carry):
            step(2 * i2, 0)
            step(2 * i2 + 1, 1)
            return carry

        n2 = count // 2
        load(0, 0)

        @pl.when(count > 1)
        def _():
            load(1, 1)

        lax.fori_loop(0, n2, body, 0)

        @pl.when(count != 2 * n2)
        def _():
            step(count - 1, 0)

        @pl.when(count >= 1)
        def _():
            wait_write(0)

        @pl.when(count >= 2)
        def _():
            wait_write(1)

        if tail:
            @pl.when(wid == NUM_WORKERS - 1)
            def _():
                tok0 = n_full * TBLK
                pltpu.sync_copy(ttail_hbm, tail_v)
                for l0 in range(0, tail, 16):
                    lvec = l0 + lax.iota(jnp.int32, 16)
                    row_idx = lax.shift_right_logical(lvec, 1)
                    col_base = (lvec & 1) * dim

                    base = row_idx * lanes + col_base

                    def cbody(c, carry):
                        v = tail_v[c, pl.ds(l0, 16)]
                        cvec = jnp.full((16,), 0, jnp.int32) + c
                        plsc.store_scatter(tp0, [base + cvec], v)
                        return carry

                    lax.fori_loop(0, dim, cbody, 0)
                pltpu.sync_copy(
                    tp0.at[pl.ds(0, tail // 2 * lanes)],
                    out_hbm.at[pl.ds(tok0 // 2 * lanes, tail // 2 * lanes)],
                )

    return k(table_v4, table_tail)


@functools.partial(jax.jit, static_argnames=("batch", "seq", "dim"))
def _sc_gather(ids, table, *, batch, seq, dim):
    rows_per_w = batch // NUM_WORKERS
    n_groups = rows_per_w // K

    mesh = plsc.VectorSubcoreMesh(core_axis_name="c", subcore_axis_name="s")

    @functools.partial(
        pl.kernel,
        out_type=jax.ShapeDtypeStruct((batch * seq, 2, dim), jnp.float32),
        mesh=mesh,
        scratch_types=[
            pltpu.VMEM((K, seq), jnp.int32),
            pltpu.VMEM((K, seq, dim), jnp.float32),
            pltpu.SemaphoreType.DMA((K,)),
            pltpu.SemaphoreType.DMA((K,)),
            pltpu.SemaphoreType.DMA((K,)),
        ],
        compiler_params=pltpu.CompilerParams(use_tc_tiling_on_sc=False),
    )
    def k(ids_hbm, table_hbm, out_hbm, idx_v, rows_v, isem, gsem, ssem):
        wid = lax.axis_index("s") * 2 + lax.axis_index("c")
        b0 = wid * rows_per_w

        def body(g, carry):
            r0 = b0 + g * K
            for b in range(K):
                pltpu.async_copy(ids_hbm.at[r0 + b], idx_v.at[b], isem.at[b])
            for b in range(K):
                pltpu.make_async_copy(
                    ids_hbm.at[r0 + b], idx_v.at[b], isem.at[b]
                ).wait()
                pltpu.async_copy(
                    table_hbm.at[idx_v.at[b]], rows_v.at[b], gsem.at[b]
                )
            for b in range(K):
                pltpu.make_async_copy(
                    table_hbm.at[idx_v.at[b]], rows_v.at[b], gsem.at[b]
                ).wait()
                pltpu.async_copy(
                    rows_v.at[b],
                    out_hbm.at[pl.ds((r0 + b) * seq, seq), 0, :],
                    ssem.at[b],
                )
            for b in range(K):
                pltpu.make_async_copy(
                    rows_v.at[b],
                    out_hbm.at[pl.ds((r0 + b) * seq, seq), 0, :],
                    ssem.at[b],
                ).wait()
            return carry

        lax.fori_loop(0, n_groups, body, 0)

    return k(ids, table)


def kernel(input_ids, token_embed):
    batch, seq = input_ids.shape
    vocab, dim = token_embed.shape
    out2 = _sc_gather(input_ids, token_embed, batch=batch, seq=seq, dim=dim)
    return out2.reshape(batch, seq, 2 * dim)[..., :dim]
